# baseline (speedup 1.0000x reference)
import jax
import jax.numpy as jnp
from jax import lax
from jax.experimental import pallas as pl
from jax.experimental.pallas import tpu as pltpu
from jax.experimental.pallas import tpu_sc as plsc

_NF = 64
_NS = 100000
_B = 1024
_TEMP = 0.05
_LOG2E = 1.4426950408889634
_LN2 = 0.6931471805599453
_SCALE = _LOG2E / _TEMP
_BN = 2000

_NW = 32
_BPW = _B // _NW


def _sc_gather_body(table_hbm, idx_hbm, out_hbm, idx_v, rows_v, sem):
    wid = lax.axis_index("s") * 2 + lax.axis_index("c")
    base = wid * _BPW
    pltpu.sync_copy(idx_hbm.at[pl.ds(base, _BPW)], idx_v)
    copies = []
    for j in range(_BPW):
        grp = idx_v[pl.ds((j // 16) * 16, 16)]
        row = grp[j % 16]
        copies.append(pltpu.async_copy(
            table_hbm.at[pl.ds(row, 1)], rows_v.at[pl.ds(j, 1)], sem))
    for c in copies:
        c.wait()
    pltpu.sync_copy(rows_v, out_hbm.at[pl.ds(base, _BPW)])


def _sc_gather(features, targets):
    mesh = plsc.VectorSubcoreMesh(core_axis_name="c", subcore_axis_name="s")
    k = pl.kernel(
        _sc_gather_body,
        mesh=mesh,
        out_type=jax.ShapeDtypeStruct((_B, _NF), jnp.float32),
        scratch_types=[
            pltpu.VMEM((_BPW,), jnp.int32),
            pltpu.VMEM((_BPW, _NF), jnp.float32),
            pltpu.SemaphoreType.DMA,
        ],
        compiler_params=pltpu.CompilerParams(use_tc_tiling_on_sc=True),
    )
    return k(features, targets)


def _sumexp_body(x_ref, f_ref, s_ref, s_acc):
    i = pl.program_id(0)

    @pl.when(i == 0)
    def _init():
        s_acc[...] = jnp.zeros_like(s_acc)

    x = x_ref[...]
    norm = jnp.sqrt(jnp.sum(x * x, axis=1, keepdims=True))
    xn = x * (_SCALE / jnp.maximum(norm, 1e-12))

    z = jax.lax.dot_general(
        xn.astype(jnp.bfloat16), f_ref[...].astype(jnp.bfloat16),
        (((1,), (1,)), ((), ())),
        preferred_element_type=jnp.float32)
    s_acc[...] += jnp.sum(jnp.exp2(z), axis=1, keepdims=True)

    @pl.when(i == pl.num_programs(0) - 1)
    def _final():
        s_ref[...] = s_acc[...]


def _combine_body(x_ref, trow_ref, s_ref, out_ref):
    x = x_ref[...]
    norm = jnp.sqrt(jnp.sum(x * x, axis=1, keepdims=True))
    xn = x * (_SCALE / jnp.maximum(norm, 1e-12))
    tgt = jnp.sum(xn * trow_ref[...], axis=1, keepdims=True)
    lse2 = jnp.log2(s_ref[...])
    out_ref[...] = (_LN2 * jnp.mean(lse2 - tgt)).reshape(1, 1)


def kernel(inputs, targets, features):
    trows = _sc_gather(features, targets.astype(jnp.int32))
    s = pl.pallas_call(
        _sumexp_body,
        grid=(_NS // _BN,),
        in_specs=[
            pl.BlockSpec((_B, _NF), lambda i: (0, 0)),
            pl.BlockSpec((_BN, _NF), lambda i: (i, 0)),
        ],
        out_specs=pl.BlockSpec((_B, 1), lambda i: (0, 0)),
        out_shape=jax.ShapeDtypeStruct((_B, 1), jnp.float32),
        scratch_shapes=[
            pltpu.VMEM((_B, 1), jnp.float32),
        ],
        compiler_params=pltpu.CompilerParams(
            dimension_semantics=("arbitrary",)),
    )(inputs, features)
    out = pl.pallas_call(
        _combine_body,
        out_shape=jax.ShapeDtypeStruct((1, 1), jnp.float32),
    )(inputs, trows, s)
    return out[0, 0]


# speedup vs baseline: 1.0810x; 1.0810x over previous
import jax
import jax.numpy as jnp
from jax import lax
from jax.experimental import pallas as pl
from jax.experimental.pallas import tpu as pltpu
from jax.experimental.pallas import tpu_sc as plsc

_NF = 64
_NS = 100000
_B = 1024
_TEMP = 0.05
_LOG2E = 1.4426950408889634
_LN2 = 0.6931471805599453
_SCALE = _LOG2E / _TEMP
_BN = 4000

_NW = 32
_BPW = _B // _NW


def _sc_gather_body(table_hbm, idx_hbm, out_hbm, idx_v, rows_v, sem):
    wid = lax.axis_index("s") * 2 + lax.axis_index("c")
    base = wid * _BPW
    pltpu.sync_copy(idx_hbm.at[pl.ds(base, _BPW)], idx_v)
    copies = []
    for j in range(_BPW):
        grp = idx_v[pl.ds((j // 16) * 16, 16)]
        row = grp[j % 16]
        copies.append(pltpu.async_copy(
            table_hbm.at[pl.ds(row, 1)], rows_v.at[pl.ds(j, 1)], sem))
    for c in copies:
        c.wait()
    pltpu.sync_copy(rows_v, out_hbm.at[pl.ds(base, _BPW)])


def _sc_gather(features, targets):
    mesh = plsc.VectorSubcoreMesh(core_axis_name="c", subcore_axis_name="s")
    k = pl.kernel(
        _sc_gather_body,
        mesh=mesh,
        out_type=jax.ShapeDtypeStruct((_B, _NF), jnp.float32),
        scratch_types=[
            pltpu.VMEM((_BPW,), jnp.int32),
            pltpu.VMEM((_BPW, _NF), jnp.float32),
            pltpu.SemaphoreType.DMA,
        ],
        compiler_params=pltpu.CompilerParams(use_tc_tiling_on_sc=True),
    )
    return k(features, targets)


def _sumexp_body(x_ref, f_ref, s_ref, s_acc):
    i = pl.program_id(0)

    @pl.when(i == 0)
    def _init():
        s_acc[...] = jnp.zeros_like(s_acc)

    x = x_ref[...]
    norm = jnp.sqrt(jnp.sum(x * x, axis=1, keepdims=True))
    xn = x * (_SCALE / jnp.maximum(norm, 1e-12))

    z = jax.lax.dot_general(
        xn.astype(jnp.bfloat16), f_ref[...].astype(jnp.bfloat16),
        (((1,), (1,)), ((), ())),
        preferred_element_type=jnp.float32)
    s_acc[...] += jnp.sum(jnp.exp2(z), axis=1, keepdims=True)

    @pl.when(i == pl.num_programs(0) - 1)
    def _final():
        s_ref[...] = s_acc[...]


def _combine_body(x_ref, trow_ref, s_ref, out_ref):
    x = x_ref[...]
    norm = jnp.sqrt(jnp.sum(x * x, axis=1, keepdims=True))
    xn = x * (_SCALE / jnp.maximum(norm, 1e-12))
    tgt = jnp.sum(xn * trow_ref[...], axis=1, keepdims=True)
    lse2 = jnp.log2(s_ref[...])
    out_ref[...] = (_LN2 * jnp.mean(lse2 - tgt)).reshape(1, 1)


def kernel(inputs, targets, features):
    trows = _sc_gather(features, targets.astype(jnp.int32))
    s = pl.pallas_call(
        _sumexp_body,
        grid=(_NS // _BN,),
        in_specs=[
            pl.BlockSpec((_B, _NF), lambda i: (0, 0)),
            pl.BlockSpec((_BN, _NF), lambda i: (i, 0)),
        ],
        out_specs=pl.BlockSpec((_B, 1), lambda i: (0, 0)),
        out_shape=jax.ShapeDtypeStruct((_B, 1), jnp.float32),
        scratch_shapes=[
            pltpu.VMEM((_B, 1), jnp.float32),
        ],
        compiler_params=pltpu.CompilerParams(
            dimension_semantics=("arbitrary",)),
    )(inputs, features)
    out = pl.pallas_call(
        _combine_body,
        out_shape=jax.ShapeDtypeStruct((1, 1), jnp.float32),
    )(inputs, trows, s)
    return out[0, 0]


# speedup vs baseline: 1.1008x; 1.0183x over previous
import jax
import jax.numpy as jnp
from jax import lax
from jax.experimental import pallas as pl
from jax.experimental.pallas import tpu as pltpu
from jax.experimental.pallas import tpu_sc as plsc

_NF = 64
_NS = 100000
_B = 1024
_TEMP = 0.05
_LOG2E = 1.4426950408889634
_LN2 = 0.6931471805599453
_SCALE = _LOG2E / _TEMP
_BN = 5000

_NW = 32
_BPW = _B // _NW


def _sc_gather_body(table_hbm, idx_hbm, out_hbm, idx_v, rows_v, sem):
    wid = lax.axis_index("s") * 2 + lax.axis_index("c")
    base = wid * _BPW
    pltpu.sync_copy(idx_hbm.at[pl.ds(base, _BPW)], idx_v)
    copies = []
    for j in range(_BPW):
        grp = idx_v[pl.ds((j // 16) * 16, 16)]
        row = grp[j % 16]
        copies.append(pltpu.async_copy(
            table_hbm.at[pl.ds(row, 1)], rows_v.at[pl.ds(j, 1)], sem))
    for c in copies:
        c.wait()
    pltpu.sync_copy(rows_v, out_hbm.at[pl.ds(base, _BPW)])


def _sc_gather(features, targets):
    mesh = plsc.VectorSubcoreMesh(core_axis_name="c", subcore_axis_name="s")
    k = pl.kernel(
        _sc_gather_body,
        mesh=mesh,
        out_type=jax.ShapeDtypeStruct((_B, _NF), jnp.float32),
        scratch_types=[
            pltpu.VMEM((_BPW,), jnp.int32),
            pltpu.VMEM((_BPW, _NF), jnp.float32),
            pltpu.SemaphoreType.DMA,
        ],
        compiler_params=pltpu.CompilerParams(use_tc_tiling_on_sc=True),
    )
    return k(features, targets)


def _sumexp_body(x_ref, f_ref, s_ref, s_acc):
    i = pl.program_id(0)

    @pl.when(i == 0)
    def _init():
        s_acc[...] = jnp.zeros_like(s_acc)

    x = x_ref[...]
    norm = jnp.sqrt(jnp.sum(x * x, axis=1, keepdims=True))
    xn = x * (_SCALE / jnp.maximum(norm, 1e-12))

    z = jax.lax.dot_general(
        xn.astype(jnp.bfloat16), f_ref[...].astype(jnp.bfloat16),
        (((1,), (1,)), ((), ())),
        preferred_element_type=jnp.float32)
    s_acc[...] += jnp.sum(jnp.exp2(z), axis=1, keepdims=True)

    @pl.when(i == pl.num_programs(0) - 1)
    def _final():
        s_ref[...] = s_acc[...]


def _combine_body(x_ref, trow_ref, s_ref, out_ref):
    x = x_ref[...]
    norm = jnp.sqrt(jnp.sum(x * x, axis=1, keepdims=True))
    xn = x * (_SCALE / jnp.maximum(norm, 1e-12))
    tgt = jnp.sum(xn * trow_ref[...], axis=1, keepdims=True)
    lse2 = jnp.log2(s_ref[...])
    out_ref[...] = (_LN2 * jnp.mean(lse2 - tgt)).reshape(1, 1)


def kernel(inputs, targets, features):
    trows = _sc_gather(features, targets.astype(jnp.int32))
    s = pl.pallas_call(
        _sumexp_body,
        grid=(_NS // _BN,),
        in_specs=[
            pl.BlockSpec((_B, _NF), lambda i: (0, 0)),
            pl.BlockSpec((_BN, _NF), lambda i: (i, 0)),
        ],
        out_specs=pl.BlockSpec((_B, 1), lambda i: (0, 0)),
        out_shape=jax.ShapeDtypeStruct((_B, 1), jnp.float32),
        scratch_shapes=[
            pltpu.VMEM((_B, 1), jnp.float32),
        ],
        compiler_params=pltpu.CompilerParams(
            dimension_semantics=("arbitrary",)),
    )(inputs, features)
    out = pl.pallas_call(
        _combine_body,
        out_shape=jax.ShapeDtypeStruct((1, 1), jnp.float32),
    )(inputs, trows, s)
    return out[0, 0]


# speedup vs baseline: 1.1257x; 1.0226x over previous
import jax
import jax.numpy as jnp
from jax import lax
from jax.experimental import pallas as pl
from jax.experimental.pallas import tpu as pltpu
from jax.experimental.pallas import tpu_sc as plsc

_NF = 64
_NS = 100000
_B = 1024
_TEMP = 0.05
_LOG2E = 1.4426950408889634
_LN2 = 0.6931471805599453
_SCALE = _LOG2E / _TEMP
_BN = 5000

_NW = 32
_BPW = _B // _NW


def _sc_gather_body(table_hbm, idx_hbm, out_hbm, idx_v, rows_v, sem):
    wid = lax.axis_index("s") * 2 + lax.axis_index("c")
    base = wid * _BPW
    pltpu.sync_copy(idx_hbm.at[pl.ds(base, _BPW)], idx_v)
    copies = []
    for j in range(_BPW):
        grp = idx_v[pl.ds((j // 16) * 16, 16)]
        row = grp[j % 16]
        copies.append(pltpu.async_copy(
            table_hbm.at[pl.ds(row, 1)], rows_v.at[pl.ds(j, 1)], sem))
    for c in copies:
        c.wait()
    pltpu.sync_copy(rows_v, out_hbm.at[pl.ds(base, _BPW)])


def _sc_gather(features, targets):
    mesh = plsc.VectorSubcoreMesh(core_axis_name="c", subcore_axis_name="s")
    k = pl.kernel(
        _sc_gather_body,
        mesh=mesh,
        out_type=jax.ShapeDtypeStruct((_B, _NF), jnp.float32),
        scratch_types=[
            pltpu.VMEM((_BPW,), jnp.int32),
            pltpu.VMEM((_BPW, _NF), jnp.float32),
            pltpu.SemaphoreType.DMA,
        ],
        compiler_params=pltpu.CompilerParams(use_tc_tiling_on_sc=True),
    )
    return k(features, targets)


def _sumexp_body(x_ref, f_ref, s_ref, s_acc):
    i = pl.program_id(0)

    @pl.when(i == 0)
    def _init():
        s_acc[...] = jnp.zeros_like(s_acc)

    x = x_ref[...]
    norm = jnp.sqrt(jnp.sum(x * x, axis=1, keepdims=True))
    xn = x * (_SCALE / jnp.maximum(norm, 1e-12))

    z = jax.lax.dot_general(
        xn, f_ref[...], (((1,), (1,)), ((), ())))
    s_acc[...] += jnp.sum(jnp.exp2(z), axis=1, keepdims=True)

    @pl.when(i == pl.num_programs(0) - 1)
    def _final():
        s_ref[...] = s_acc[...]


def _combine_body(x_ref, trow_ref, s_ref, out_ref):
    x = x_ref[...]
    norm = jnp.sqrt(jnp.sum(x * x, axis=1, keepdims=True))
    xn = x * (_SCALE / jnp.maximum(norm, 1e-12))
    tgt = jnp.sum(xn * trow_ref[...], axis=1, keepdims=True)
    lse2 = jnp.log2(s_ref[...])
    out_ref[...] = (_LN2 * jnp.mean(lse2 - tgt)).reshape(1, 1)


def kernel(inputs, targets, features):
    trows = _sc_gather(features, targets.astype(jnp.int32))
    s = pl.pallas_call(
        _sumexp_body,
        grid=(_NS // _BN,),
        in_specs=[
            pl.BlockSpec((_B, _NF), lambda i: (0, 0)),
            pl.BlockSpec((_BN, _NF), lambda i: (i, 0)),
        ],
        out_specs=pl.BlockSpec((_B, 1), lambda i: (0, 0)),
        out_shape=jax.ShapeDtypeStruct((_B, 1), jnp.float32),
        scratch_shapes=[
            pltpu.VMEM((_B, 1), jnp.float32),
        ],
        compiler_params=pltpu.CompilerParams(
            dimension_semantics=("arbitrary",)),
    )(inputs, features)
    out = pl.pallas_call(
        _combine_body,
        out_shape=jax.ShapeDtypeStruct((1, 1), jnp.float32),
    )(inputs, trows, s)
    return out[0, 0]
